# baseline (speedup 1.0000x reference)
import functools
import math

import jax
import jax.numpy as jnp
from jax import lax
from jax.experimental import pallas as pl
from jax.experimental.pallas import tpu as pltpu
from jax.experimental.pallas import tpu_sc as plsc

F32 = jnp.float32
BF16 = jnp.bfloat16
_PREC = lax.Precision.HIGHEST

H = 128
N_NODES = 10000
N_PAD = 10112
NQ = 80
CK = 64
NC, NS = 2, 16
NW = NC * NS
ROWS_PER_TILE = N_PAD // NS
ZSLICES = tuple((i * 64, 64) for i in range(9)) + ((576, 56),)


def _silu(v):
    return v / (1.0 + jnp.exp(-v))


def _proj_body(x_ref, w_ref, o_ref):
    o_ref[...] = jnp.dot(x_ref[...], w_ref[0], preferred_element_type=F32,
                         precision=_PREC)


def _node_projections(x, w3):
    n = x.shape[0]
    bn = 2000
    nb = n // bn
    return pl.pallas_call(
        _proj_body,
        grid=(3, nb),
        in_specs=[pl.BlockSpec((bn, H), lambda i, j: (j, 0)),
                  pl.BlockSpec((1, H, H), lambda i, j: (i, 0, 0))],
        out_specs=pl.BlockSpec((bn, H), lambda i, j: (i * nb + j, 0)),
        out_shape=jax.ShapeDtypeStruct((3 * n, H), F32),
    )(x, w3)


def _edge_rbf_body(rbf_ref, w_ref, b_ref, o_ref):
    o_ref[...] = (jnp.dot(rbf_ref[...], w_ref[...], preferred_element_type=F32,
                          precision=_PREC) + b_ref[...])


def _edge_rbf_term(rbf, w, b2d):
    e, k = rbf.shape
    bn = 8000
    return pl.pallas_call(
        _edge_rbf_body,
        grid=(e // bn,),
        in_specs=[pl.BlockSpec((bn, k), lambda i: (i, 0)),
                  pl.BlockSpec((k, H), lambda i: (0, 0)),
                  pl.BlockSpec((1, H), lambda i: (0, 0))],
        out_specs=pl.BlockSpec((bn, H), lambda i: (i, 0)),
        out_shape=jax.ShapeDtypeStruct((e, H), F32),
    )(rbf, w, b2d)


def _angle_rbf_body(inv_sig2, a_ref, c_ref, w_ref, b_ref, o_ref):
    a = a_ref[...]
    c = c_ref[...]
    d = a - c
    rbf = jnp.exp(-(d * d) * inv_sig2)
    o_ref[...] = (jnp.dot(rbf, w_ref[...], preferred_element_type=F32,
                          precision=_PREC) + b_ref[...])


def _angle_rbf_term(angles2d, centers2d, w, b2d):
    m = angles2d.shape[0]
    k = centers2d.shape[1]
    sigma = math.pi / k
    bn = 8000
    return pl.pallas_call(
        functools.partial(_angle_rbf_body, 1.0 / (sigma * sigma)),
        grid=(m // bn,),
        in_specs=[pl.BlockSpec((bn, 1), lambda i: (i, 0)),
                  pl.BlockSpec((1, k), lambda i: (0, 0)),
                  pl.BlockSpec((k, H), lambda i: (0, 0)),
                  pl.BlockSpec((1, H), lambda i: (0, 0))],
        out_specs=pl.BlockSpec((bn, H), lambda i: (i, 0)),
        out_shape=jax.ShapeDtypeStruct((m, H), F32),
    )(angles2d, centers2d, w, b2d)


def _count_body(idxr_ref, idxc_ref, o_ref):
    i = pl.program_id(0)
    q = idxc_ref[...] >> 7
    r = idxr_ref[...] & 127
    ioq = lax.broadcasted_iota(jnp.int32, (NQ, 1), 0)
    ior = lax.broadcasted_iota(jnp.int32, (1, H), 1)
    ohq = (q == ioq).astype(BF16)
    ohr = (r == ior).astype(BF16)
    cblk = jnp.dot(ohq, ohr, preferred_element_type=F32)

    @pl.when(i == 0)
    def _init():
        o_ref[...] = cblk

    @pl.when(i > 0)
    def _accum():
        o_ref[...] += cblk


def _counts(idx):
    n = idx.shape[0]
    bn = 12800
    return pl.pallas_call(
        _count_body,
        grid=(n // bn,),
        in_specs=[pl.BlockSpec((bn, 1), lambda i: (i, 0)),
                  pl.BlockSpec((1, bn), lambda i: (0, i))],
        out_specs=pl.BlockSpec((NQ, H), lambda i: (0, 0)),
        out_shape=jax.ShapeDtypeStruct((NQ, H), F32),
    )(idx.reshape(n, 1), idx.reshape(1, n))


def _sc_body(p_hbm, row_hbm, col_hbm, cen_hbm, be_hbm, bt_hbm,
             oute_hbm, outt_hbm,
             row0_v, row1_v, col0_v, col1_v,
             g10_v, g11_v, g20_v, g21_v, b0_v, b1_v,
             acc, sg10, sg11, sg20, sg21, sb0, sb1, sc0, sc1):
    c = lax.axis_index("c")
    s = lax.axis_index("s")
    w = s * NC + c

    row_v = (row0_v, row1_v)
    col_v = (col0_v, col1_v)
    g1_v = (g10_v, g11_v)
    g2_v = (g20_v, g21_v)
    b_v = (b0_v, b1_v)
    sg1 = (sg10, sg11)
    sg2 = (sg20, sg21)
    sb = (sb0, sb1)
    sc = (sc0, sc1)

    n_echunks = row_hbm.shape[0]
    n_tchunks = cen_hbm.shape[0]
    n_e = (n_echunks - 1 - w) // NW + 1
    n_t = (n_tchunks - 1 - w) // NW + 1

    def _zero_g1():
        def _zrow(r, _):
            for blk in range(H // 16):
                g10_v[r, pl.ds(16 * blk, 16)] = jnp.zeros((16,), F32)
            return _
        lax.fori_loop(0, CK, _zrow, None)

    _zero_g1()
    for off, sz in ZSLICES:
        pltpu.sync_copy(g10_v.at[pl.ds(0, sz)],
                        acc.at[pl.ds(s * ROWS_PER_TILE + off, sz)])
    plsc.subcore_barrier()

    def _eissue(k, i):
        ch = w + i * NW
        pltpu.sync_copy(row_hbm.at[ch], row_v[k])
        pltpu.sync_copy(col_hbm.at[ch], col_v[k])
        pltpu.async_copy(p_hbm.at[row_v[k].at[0]], g1_v[k], sg1[k])
        pltpu.async_copy(p_hbm.at[col_v[k].at[0]], g2_v[k], sg2[k])
        pltpu.async_copy(be_hbm.at[ch], b_v[k], sb[k])

    def _edrain(k):
        pltpu.make_async_copy(p_hbm.at[pl.ds(0, CK)], g1_v[k], sg1[k]).wait()
        pltpu.make_async_copy(p_hbm.at[pl.ds(0, CK)], g2_v[k], sg2[k]).wait()
        pltpu.make_async_copy(p_hbm.at[pl.ds(0, CK)], b_v[k], sb[k]).wait()

    def _ecompute(k):
        def _crow(r2, __):
            for dr in range(2):
                r = r2 * 2 + dr
                for blk in range(H // 16):
                    sl = pl.ds(16 * blk, 16)
                    g1_v[k][r, sl] = _silu(g1_v[k][r, sl] + g2_v[k][r, sl]
                                           + b_v[k][r, sl])
            return __
        lax.fori_loop(0, CK // 2, _crow, None)
        pltpu.async_copy(g1_v[k], acc.at[row_v[k].at[0]], sc[k], add=True)

    def _scwait(k):
        pltpu.make_async_copy(g1_v[k], acc.at[pl.ds(0, CK)], sc[k]).wait()

    _eissue(0, 0)

    def _epair(j, _):
        @pl.when(2 * j + 1 < n_e)
        def _():
            @pl.when(j > 0)
            def _():
                _scwait(1)
            _eissue(1, 2 * j + 1)
        _edrain(0)
        _ecompute(0)

        @pl.when(2 * j + 1 < n_e)
        def _():
            _edrain(1)
            _ecompute(1)
        _scwait(0)

        @pl.when(2 * j + 2 < n_e)
        def _():
            _eissue(0, 2 * j + 2)
        return _
    lax.fori_loop(0, (n_e + 1) // 2, _epair, None)
    _scwait(1)
    plsc.subcore_barrier()

    _zero_g1()
    for off, sz in ZSLICES:
        base = s * ROWS_PER_TILE + off
        pltpu.sync_copy(acc.at[pl.ds(base, sz)],
                        oute_hbm.at[c, pl.ds(base, sz)])
        pltpu.sync_copy(g10_v.at[pl.ds(0, sz)], acc.at[pl.ds(base, sz)])
    plsc.subcore_barrier()

    def _tissue(k, i):
        ch = w + i * NW
        pltpu.sync_copy(cen_hbm.at[ch], row_v[k])
        for blk in range(CK // 16):
            sl = pl.ds(16 * blk, 16)
            col_v[k][0, sl] = row_v[k][0, sl] + jnp.full((16,), 2 * N_NODES,
                                                         jnp.int32)
        pltpu.async_copy(p_hbm.at[col_v[k].at[0]], g1_v[k], sg1[k])
        pltpu.async_copy(bt_hbm.at[ch], b_v[k], sb[k])

    def _tdrain(k):
        pltpu.make_async_copy(p_hbm.at[pl.ds(0, CK)], g1_v[k], sg1[k]).wait()
        pltpu.make_async_copy(p_hbm.at[pl.ds(0, CK)], b_v[k], sb[k]).wait()

    def _tcompute(k):
        def _crow(r2, __):
            for dr in range(2):
                r = r2 * 2 + dr
                for blk in range(H // 16):
                    sl = pl.ds(16 * blk, 16)
                    g1_v[k][r, sl] = _silu(g1_v[k][r, sl] + b_v[k][r, sl])
            return __
        lax.fori_loop(0, CK // 2, _crow, None)
        pltpu.async_copy(g1_v[k], acc.at[row_v[k].at[0]], sc[k], add=True)

    _tissue(0, 0)

    def _tpair(j, _):
        @pl.when(2 * j + 1 < n_t)
        def _():
            @pl.when(j > 0)
            def _():
                _scwait(1)
            _tissue(1, 2 * j + 1)
        _tdrain(0)
        _tcompute(0)

        @pl.when(2 * j + 1 < n_t)
        def _():
            _tdrain(1)
            _tcompute(1)
        _scwait(0)

        @pl.when(2 * j + 2 < n_t)
        def _():
            _tissue(0, 2 * j + 2)
        return _
    lax.fori_loop(0, (n_t + 1) // 2, _tpair, None)
    _scwait(1)
    plsc.subcore_barrier()

    for off, sz in ZSLICES:
        base = s * ROWS_PER_TILE + off
        pltpu.sync_copy(acc.at[pl.ds(base, sz)],
                        outt_hbm.at[c, pl.ds(base, sz)])


def _sc_aggregate(p, row3, col3, cen3, be3, bt3):
    mesh = plsc.VectorSubcoreMesh(core_axis_name="c", subcore_axis_name="s")
    outh = jax.ShapeDtypeStruct((NC, N_PAD, H), F32)
    run = pl.kernel(
        _sc_body,
        mesh=mesh,
        out_type=[outh, outh],
        scratch_types=(
            [pltpu.VMEM((1, CK), jnp.int32)] * 4
            + [pltpu.VMEM((CK, H), F32)] * 6
            + [pltpu.VMEM_SHARED((N_PAD, H), F32)]
            + [pltpu.SemaphoreType.DMA] * 8
        ),
    )
    return run(p, row3, col3, cen3, be3, bt3)


def _combine_body(x_ref, se_ref, st_ref, ce_ref, ct_ref,
                  we2_ref, be2_ref, wt2_ref, bt2_ref,
                  wn1a_ref, wn1b_ref, bn1_ref, wn2_ref, bn2_ref, o_ref):
    xb = x_ref[...]
    se = se_ref[0] + se_ref[1]
    st = st_ref[0] + st_ref[1]
    aggr = (jnp.dot(se, we2_ref[...], preferred_element_type=F32, precision=_PREC)
            + ce_ref[...] * be2_ref[...]
            + jnp.dot(st, wt2_ref[...], preferred_element_type=F32, precision=_PREC)
            + ct_ref[...] * bt2_ref[...])
    h1 = (jnp.dot(xb, wn1a_ref[...], preferred_element_type=F32, precision=_PREC)
          + jnp.dot(aggr, wn1b_ref[...], preferred_element_type=F32, precision=_PREC)
          + bn1_ref[...])
    h1 = _silu(h1)
    o_ref[...] = xb + jnp.dot(h1, wn2_ref[...], preferred_element_type=F32,
                              precision=_PREC) + bn2_ref[...]


def _combine(x, se, st, ce, ct, we2, be2, wt2, bt2, wn1a, wn1b, bn1, wn2, bn2):
    n = x.shape[0]
    bn = 2000
    full = lambda r, c: pl.BlockSpec((r, c), lambda i: (0, 0))
    acc_spec = pl.BlockSpec((NC, bn, H), lambda i: (0, i, 0))
    cnt_spec = pl.BlockSpec((bn, 1), lambda i: (i, 0))
    return pl.pallas_call(
        _combine_body,
        grid=(n // bn,),
        in_specs=[pl.BlockSpec((bn, H), lambda i: (i, 0)),
                  acc_spec, acc_spec, cnt_spec, cnt_spec,
                  full(H, H), full(1, H), full(H, H), full(1, H),
                  full(H, H), full(H, H), full(1, H), full(H, H), full(1, H)],
        out_specs=pl.BlockSpec((bn, H), lambda i: (i, 0)),
        out_shape=jax.ShapeDtypeStruct((n, H), F32),
    )(x, se, st, ce, ct, we2, be2, wt2, bt2, wn1a, wn1b, bn1, wn2, bn2)


def kernel(x, edge_index, edge_attr_rbf, triplet_index, angles,
           W_e1, b_e1, W_e2, b_e2,
           W_t1, b_t1, W_t2, b_t2,
           W_n1, b_n1, W_n2, b_n2,
           centers):
    e = edge_index.shape[1]
    m = triplet_index.shape[0]
    k = centers.shape[0]

    w3 = jnp.stack([W_e1[:H], W_e1[H:2 * H], W_t1[:H]])
    p = _node_projections(x, w3)
    be = _edge_rbf_term(edge_attr_rbf, W_e1[2 * H:], b_e1.reshape(1, H))
    bt = _angle_rbf_term(angles.reshape(m, 1), centers.reshape(1, k),
                         W_t1[H:], b_t1.reshape(1, H))

    row = edge_index[0]
    cen = triplet_index[:, 1]
    cnt_e = _counts(row).reshape(NQ * H, 1)
    cnt_t = _counts(cen).reshape(NQ * H, 1)

    n = x.shape[0]
    row3 = row.reshape(e // CK, 1, CK)
    col3 = (edge_index[1] + n).reshape(e // CK, 1, CK)
    cen3 = cen.reshape(m // CK, 1, CK)
    be3 = be.reshape(e // CK, CK, H)
    bt3 = bt.reshape(m // CK, CK, H)
    se, st = _sc_aggregate(p, row3, col3, cen3, be3, bt3)

    return _combine(x, se, st, cnt_e[:N_NODES], cnt_t[:N_NODES],
                    W_e2, b_e2.reshape(1, H), W_t2, b_t2.reshape(1, H),
                    W_n1[:H], W_n1[H:], b_n1.reshape(1, H), W_n2, b_n2.reshape(1, H))


# speedup vs baseline: 1.0702x; 1.0702x over previous
import functools
import math

import jax
import jax.numpy as jnp
from jax import lax
from jax.experimental import pallas as pl
from jax.experimental.pallas import tpu as pltpu
from jax.experimental.pallas import tpu_sc as plsc

F32 = jnp.float32
BF16 = jnp.bfloat16
_PREC = lax.Precision.HIGHEST

H = 128
N_NODES = 10000
N_PAD = 10112
NQ = 80
CK = 64
NC, NS = 2, 16
NW = NC * NS
ROWS_PER_TILE = N_PAD // NS
ZSLICES = tuple((i * 64, 64) for i in range(9)) + ((576, 56),)


def _silu(v):
    return v / (1.0 + jnp.exp(-v))


def _proj_body(x_ref, w_ref, o_ref):
    o_ref[...] = jnp.dot(x_ref[...], w_ref[0], preferred_element_type=F32,
                         precision=_PREC)


def _node_projections(x, w3):
    n = x.shape[0]
    bn = 2000
    nb = n // bn
    return pl.pallas_call(
        _proj_body,
        grid=(3, nb),
        in_specs=[pl.BlockSpec((bn, H), lambda i, j: (j, 0)),
                  pl.BlockSpec((1, H, H), lambda i, j: (i, 0, 0))],
        out_specs=pl.BlockSpec((bn, H), lambda i, j: (i * nb + j, 0)),
        out_shape=jax.ShapeDtypeStruct((3 * n, H), F32),
    )(x, w3)


def _edge_rbf_body(rbf_ref, w_ref, b_ref, o_ref):
    o_ref[...] = (jnp.dot(rbf_ref[...], w_ref[...], preferred_element_type=F32,
                          precision=_PREC) + b_ref[...])


def _edge_rbf_term(rbf, w, b2d):
    e, k = rbf.shape
    bn = 8000
    return pl.pallas_call(
        _edge_rbf_body,
        grid=(e // bn,),
        in_specs=[pl.BlockSpec((bn, k), lambda i: (i, 0)),
                  pl.BlockSpec((k, H), lambda i: (0, 0)),
                  pl.BlockSpec((1, H), lambda i: (0, 0))],
        out_specs=pl.BlockSpec((bn, H), lambda i: (i, 0)),
        out_shape=jax.ShapeDtypeStruct((e, H), F32),
    )(rbf, w, b2d)


def _angle_rbf_body(inv_sig2, a_ref, c_ref, w_ref, b_ref, o_ref):
    a = a_ref[...]
    c = c_ref[...]
    d = a - c
    rbf = jnp.exp(-(d * d) * inv_sig2)
    o_ref[...] = (jnp.dot(rbf, w_ref[...], preferred_element_type=F32,
                          precision=_PREC) + b_ref[...])


def _angle_rbf_term(angles2d, centers2d, w, b2d):
    m = angles2d.shape[0]
    k = centers2d.shape[1]
    sigma = math.pi / k
    bn = 8000
    return pl.pallas_call(
        functools.partial(_angle_rbf_body, 1.0 / (sigma * sigma)),
        grid=(m // bn,),
        in_specs=[pl.BlockSpec((bn, 1), lambda i: (i, 0)),
                  pl.BlockSpec((1, k), lambda i: (0, 0)),
                  pl.BlockSpec((k, H), lambda i: (0, 0)),
                  pl.BlockSpec((1, H), lambda i: (0, 0))],
        out_specs=pl.BlockSpec((bn, H), lambda i: (i, 0)),
        out_shape=jax.ShapeDtypeStruct((m, H), F32),
    )(angles2d, centers2d, w, b2d)


def _count_body(idxr_ref, idxc_ref, o_ref):
    i = pl.program_id(0)
    q = idxc_ref[...] >> 7
    r = idxr_ref[...] & 127
    ioq = lax.broadcasted_iota(jnp.int32, (NQ, 1), 0)
    ior = lax.broadcasted_iota(jnp.int32, (1, H), 1)
    ohq = (q == ioq).astype(BF16)
    ohr = (r == ior).astype(BF16)
    cblk = jnp.dot(ohq, ohr, preferred_element_type=F32)

    @pl.when(i == 0)
    def _init():
        o_ref[...] = cblk

    @pl.when(i > 0)
    def _accum():
        o_ref[...] += cblk


def _counts(idx):
    n = idx.shape[0]
    bn = 12800
    return pl.pallas_call(
        _count_body,
        grid=(n // bn,),
        in_specs=[pl.BlockSpec((bn, 1), lambda i: (i, 0)),
                  pl.BlockSpec((1, bn), lambda i: (0, i))],
        out_specs=pl.BlockSpec((NQ, H), lambda i: (0, 0)),
        out_shape=jax.ShapeDtypeStruct((NQ, H), F32),
    )(idx.reshape(n, 1), idx.reshape(1, n))


def _sc_body(p_hbm, row_hbm, col_hbm, cen_hbm, be_hbm, bt_hbm,
             oute_hbm, outt_hbm,
             row0_v, row1_v, col0_v, col1_v,
             rp0_v, rp1_v, cp0_v, cp1_v,
             g10_v, g11_v, g20_v, g21_v, b0_v, b1_v,
             acc, sg10, sg11, sg20, sg21, sb0, sb1, sc0, sc1, si0, si1):
    c = lax.axis_index("c")
    s = lax.axis_index("s")
    w = s * NC + c

    row_v = (row0_v, row1_v)
    col_v = (col0_v, col1_v)
    rp_v = (rp0_v, rp1_v)
    cp_v = (cp0_v, cp1_v)
    si = (si0, si1)
    g1_v = (g10_v, g11_v)
    g2_v = (g20_v, g21_v)
    b_v = (b0_v, b1_v)
    sg1 = (sg10, sg11)
    sg2 = (sg20, sg21)
    sb = (sb0, sb1)
    sc = (sc0, sc1)

    n_echunks = row_hbm.shape[0]
    n_tchunks = cen_hbm.shape[0]
    n_e = (n_echunks - 1 - w) // NW + 1
    n_t = (n_tchunks - 1 - w) // NW + 1

    def _zero_g1():
        def _zrow(r, _):
            for blk in range(H // 16):
                g10_v[r, pl.ds(16 * blk, 16)] = jnp.zeros((16,), F32)
            return _
        lax.fori_loop(0, CK, _zrow, None)

    _zero_g1()
    for off, sz in ZSLICES:
        pltpu.sync_copy(g10_v.at[pl.ds(0, sz)],
                        acc.at[pl.ds(s * ROWS_PER_TILE + off, sz)])
    plsc.subcore_barrier()

    def _eissue(k, i):
        ch = w + i * NW
        pltpu.make_async_copy(row_hbm.at[ch], rp_v[k], si[k]).wait()
        pltpu.make_async_copy(col_hbm.at[ch], cp_v[k], si[k]).wait()
        for blk in range(CK // 16):
            sl = pl.ds(16 * blk, 16)
            row_v[k][0, sl] = rp_v[k][0, sl]
            col_v[k][0, sl] = cp_v[k][0, sl]

        @pl.when(i + 2 < n_e)
        def _():
            ch2 = w + (i + 2) * NW
            pltpu.async_copy(row_hbm.at[ch2], rp_v[k], si[k])
            pltpu.async_copy(col_hbm.at[ch2], cp_v[k], si[k])
        pltpu.async_copy(p_hbm.at[row_v[k].at[0]], g1_v[k], sg1[k])
        pltpu.async_copy(p_hbm.at[col_v[k].at[0]], g2_v[k], sg2[k])
        pltpu.async_copy(be_hbm.at[ch], b_v[k], sb[k])

    def _edrain(k):
        pltpu.make_async_copy(p_hbm.at[pl.ds(0, CK)], g1_v[k], sg1[k]).wait()
        pltpu.make_async_copy(p_hbm.at[pl.ds(0, CK)], g2_v[k], sg2[k]).wait()
        pltpu.make_async_copy(p_hbm.at[pl.ds(0, CK)], b_v[k], sb[k]).wait()

    def _ecompute(k):
        def _crow(r2, __):
            for dr in range(2):
                r = r2 * 2 + dr
                for blk in range(H // 16):
                    sl = pl.ds(16 * blk, 16)
                    g1_v[k][r, sl] = _silu(g1_v[k][r, sl] + g2_v[k][r, sl]
                                           + b_v[k][r, sl])
            return __
        lax.fori_loop(0, CK // 2, _crow, None)
        pltpu.async_copy(g1_v[k], acc.at[row_v[k].at[0]], sc[k], add=True)

    def _scwait(k):
        pltpu.make_async_copy(g1_v[k], acc.at[pl.ds(0, CK)], sc[k]).wait()

    pltpu.async_copy(row_hbm.at[w], rp0_v, si0)
    pltpu.async_copy(col_hbm.at[w], cp0_v, si0)

    @pl.when(1 < n_e)
    def _():
        pltpu.async_copy(row_hbm.at[w + NW], rp1_v, si1)
        pltpu.async_copy(col_hbm.at[w + NW], cp1_v, si1)
    _eissue(0, 0)

    def _epair(j, _):
        @pl.when(2 * j + 1 < n_e)
        def _():
            @pl.when(j > 0)
            def _():
                _scwait(1)
            _eissue(1, 2 * j + 1)
        _edrain(0)
        _ecompute(0)

        @pl.when(2 * j + 1 < n_e)
        def _():
            _edrain(1)
            _ecompute(1)
        _scwait(0)

        @pl.when(2 * j + 2 < n_e)
        def _():
            _eissue(0, 2 * j + 2)
        return _
    lax.fori_loop(0, (n_e + 1) // 2, _epair, None)
    _scwait(1)
    plsc.subcore_barrier()

    _zero_g1()
    for off, sz in ZSLICES:
        base = s * ROWS_PER_TILE + off
        pltpu.sync_copy(acc.at[pl.ds(base, sz)],
                        oute_hbm.at[c, pl.ds(base, sz)])
        pltpu.sync_copy(g10_v.at[pl.ds(0, sz)], acc.at[pl.ds(base, sz)])
    plsc.subcore_barrier()

    def _tissue(k, i):
        ch = w + i * NW
        pltpu.make_async_copy(cen_hbm.at[ch], rp_v[k], si[k]).wait()
        for blk in range(CK // 16):
            sl = pl.ds(16 * blk, 16)
            row_v[k][0, sl] = rp_v[k][0, sl]
            col_v[k][0, sl] = rp_v[k][0, sl] + jnp.full((16,), 2 * N_NODES,
                                                        jnp.int32)

        @pl.when(i + 2 < n_t)
        def _():
            pltpu.async_copy(cen_hbm.at[w + (i + 2) * NW], rp_v[k], si[k])
        pltpu.async_copy(p_hbm.at[col_v[k].at[0]], g1_v[k], sg1[k])
        pltpu.async_copy(bt_hbm.at[ch], b_v[k], sb[k])

    def _tdrain(k):
        pltpu.make_async_copy(p_hbm.at[pl.ds(0, CK)], g1_v[k], sg1[k]).wait()
        pltpu.make_async_copy(p_hbm.at[pl.ds(0, CK)], b_v[k], sb[k]).wait()

    def _tcompute(k):
        def _crow(r2, __):
            for dr in range(2):
                r = r2 * 2 + dr
                for blk in range(H // 16):
                    sl = pl.ds(16 * blk, 16)
                    g1_v[k][r, sl] = _silu(g1_v[k][r, sl] + b_v[k][r, sl])
            return __
        lax.fori_loop(0, CK // 2, _crow, None)
        pltpu.async_copy(g1_v[k], acc.at[row_v[k].at[0]], sc[k], add=True)

    pltpu.async_copy(cen_hbm.at[w], rp0_v, si0)

    @pl.when(1 < n_t)
    def _():
        pltpu.async_copy(cen_hbm.at[w + NW], rp1_v, si1)
    _tissue(0, 0)

    def _tpair(j, _):
        @pl.when(2 * j + 1 < n_t)
        def _():
            @pl.when(j > 0)
            def _():
                _scwait(1)
            _tissue(1, 2 * j + 1)
        _tdrain(0)
        _tcompute(0)

        @pl.when(2 * j + 1 < n_t)
        def _():
            _tdrain(1)
            _tcompute(1)
        _scwait(0)

        @pl.when(2 * j + 2 < n_t)
        def _():
            _tissue(0, 2 * j + 2)
        return _
    lax.fori_loop(0, (n_t + 1) // 2, _tpair, None)
    _scwait(1)
    plsc.subcore_barrier()

    for off, sz in ZSLICES:
        base = s * ROWS_PER_TILE + off
        pltpu.sync_copy(acc.at[pl.ds(base, sz)],
                        outt_hbm.at[c, pl.ds(base, sz)])


def _sc_aggregate(p, row3, col3, cen3, be3, bt3):
    mesh = plsc.VectorSubcoreMesh(core_axis_name="c", subcore_axis_name="s")
    outh = jax.ShapeDtypeStruct((NC, N_PAD, H), F32)
    run = pl.kernel(
        _sc_body,
        mesh=mesh,
        out_type=[outh, outh],
        scratch_types=(
            [pltpu.VMEM((1, CK), jnp.int32)] * 8
            + [pltpu.VMEM((CK, H), F32)] * 6
            + [pltpu.VMEM_SHARED((N_PAD, H), F32)]
            + [pltpu.SemaphoreType.DMA] * 10
        ),
    )
    return run(p, row3, col3, cen3, be3, bt3)


def _combine_body(x_ref, se_ref, st_ref, ce_ref, ct_ref,
                  we2_ref, be2_ref, wt2_ref, bt2_ref,
                  wn1a_ref, wn1b_ref, bn1_ref, wn2_ref, bn2_ref, o_ref):
    xb = x_ref[...]
    se = se_ref[0] + se_ref[1]
    st = st_ref[0] + st_ref[1]
    aggr = (jnp.dot(se, we2_ref[...], preferred_element_type=F32, precision=_PREC)
            + ce_ref[...] * be2_ref[...]
            + jnp.dot(st, wt2_ref[...], preferred_element_type=F32, precision=_PREC)
            + ct_ref[...] * bt2_ref[...])
    h1 = (jnp.dot(xb, wn1a_ref[...], preferred_element_type=F32, precision=_PREC)
          + jnp.dot(aggr, wn1b_ref[...], preferred_element_type=F32, precision=_PREC)
          + bn1_ref[...])
    h1 = _silu(h1)
    o_ref[...] = xb + jnp.dot(h1, wn2_ref[...], preferred_element_type=F32,
                              precision=_PREC) + bn2_ref[...]


def _combine(x, se, st, ce, ct, we2, be2, wt2, bt2, wn1a, wn1b, bn1, wn2, bn2):
    n = x.shape[0]
    bn = 2000
    full = lambda r, c: pl.BlockSpec((r, c), lambda i: (0, 0))
    acc_spec = pl.BlockSpec((NC, bn, H), lambda i: (0, i, 0))
    cnt_spec = pl.BlockSpec((bn, 1), lambda i: (i, 0))
    return pl.pallas_call(
        _combine_body,
        grid=(n // bn,),
        in_specs=[pl.BlockSpec((bn, H), lambda i: (i, 0)),
                  acc_spec, acc_spec, cnt_spec, cnt_spec,
                  full(H, H), full(1, H), full(H, H), full(1, H),
                  full(H, H), full(H, H), full(1, H), full(H, H), full(1, H)],
        out_specs=pl.BlockSpec((bn, H), lambda i: (i, 0)),
        out_shape=jax.ShapeDtypeStruct((n, H), F32),
    )(x, se, st, ce, ct, we2, be2, wt2, bt2, wn1a, wn1b, bn1, wn2, bn2)


def kernel(x, edge_index, edge_attr_rbf, triplet_index, angles,
           W_e1, b_e1, W_e2, b_e2,
           W_t1, b_t1, W_t2, b_t2,
           W_n1, b_n1, W_n2, b_n2,
           centers):
    e = edge_index.shape[1]
    m = triplet_index.shape[0]
    k = centers.shape[0]

    w3 = jnp.stack([W_e1[:H], W_e1[H:2 * H], W_t1[:H]])
    p = _node_projections(x, w3)
    be = _edge_rbf_term(edge_attr_rbf, W_e1[2 * H:], b_e1.reshape(1, H))
    bt = _angle_rbf_term(angles.reshape(m, 1), centers.reshape(1, k),
                         W_t1[H:], b_t1.reshape(1, H))

    row = edge_index[0]
    cen = triplet_index[:, 1]
    cnt_e = _counts(row).reshape(NQ * H, 1)
    cnt_t = _counts(cen).reshape(NQ * H, 1)

    n = x.shape[0]
    row3 = row.reshape(e // CK, 1, CK)
    col3 = (edge_index[1] + n).reshape(e // CK, 1, CK)
    cen3 = cen.reshape(m // CK, 1, CK)
    be3 = be.reshape(e // CK, CK, H)
    bt3 = bt.reshape(m // CK, CK, H)
    se, st = _sc_aggregate(p, row3, col3, cen3, be3, bt3)

    return _combine(x, se, st, cnt_e[:N_NODES], cnt_t[:N_NODES],
                    W_e2, b_e2.reshape(1, H), W_t2, b_t2.reshape(1, H),
                    W_n1[:H], W_n1[H:], b_n1.reshape(1, H), W_n2, b_n2.reshape(1, H))


# speedup vs baseline: 1.0955x; 1.0236x over previous
import functools
import math

import jax
import jax.numpy as jnp
from jax import lax
from jax.experimental import pallas as pl
from jax.experimental.pallas import tpu as pltpu
from jax.experimental.pallas import tpu_sc as plsc

F32 = jnp.float32
BF16 = jnp.bfloat16
_PREC = lax.Precision.HIGHEST

H = 128
N_NODES = 10000
N_PAD = 10112
NQ = 80
CK = 64
NC, NS = 2, 16
NW = NC * NS
ROWS_PER_TILE = N_PAD // NS
ZSLICES = tuple((i * 64, 64) for i in range(9)) + ((576, 56),)


def _silu(v):
    return v / (1.0 + jnp.exp(-v))


def _proj_body(x_ref, w_ref, o_ref):
    o_ref[...] = jnp.dot(x_ref[...], w_ref[0], preferred_element_type=F32,
                         precision=_PREC)


def _node_projections(x, w3):
    n = x.shape[0]
    bn = 2000
    nb = n // bn
    return pl.pallas_call(
        _proj_body,
        grid=(3, nb),
        in_specs=[pl.BlockSpec((bn, H), lambda i, j: (j, 0)),
                  pl.BlockSpec((1, H, H), lambda i, j: (i, 0, 0))],
        out_specs=pl.BlockSpec((bn, H), lambda i, j: (i * nb + j, 0)),
        out_shape=jax.ShapeDtypeStruct((3 * n, H), F32),
    )(x, w3)


def _edge_rbf_body(rbf_ref, w_ref, b_ref, o_ref):
    o_ref[...] = (jnp.dot(rbf_ref[...], w_ref[...], preferred_element_type=F32,
                          precision=_PREC) + b_ref[...])


def _edge_rbf_term(rbf, w, b2d):
    e, k = rbf.shape
    bn = 8000
    return pl.pallas_call(
        _edge_rbf_body,
        grid=(e // bn,),
        in_specs=[pl.BlockSpec((bn, k), lambda i: (i, 0)),
                  pl.BlockSpec((k, H), lambda i: (0, 0)),
                  pl.BlockSpec((1, H), lambda i: (0, 0))],
        out_specs=pl.BlockSpec((bn, H), lambda i: (i, 0)),
        out_shape=jax.ShapeDtypeStruct((e, H), F32),
    )(rbf, w, b2d)


def _angle_rbf_body(inv_sig2, a_ref, c_ref, w_ref, b_ref, o_ref):
    a = a_ref[...]
    c = c_ref[...]
    d = a - c
    rbf = jnp.exp(-(d * d) * inv_sig2)
    o_ref[...] = (jnp.dot(rbf, w_ref[...], preferred_element_type=F32,
                          precision=_PREC) + b_ref[...])


def _angle_rbf_term(angles2d, centers2d, w, b2d):
    m = angles2d.shape[0]
    k = centers2d.shape[1]
    sigma = math.pi / k
    bn = 8000
    return pl.pallas_call(
        functools.partial(_angle_rbf_body, 1.0 / (sigma * sigma)),
        grid=(m // bn,),
        in_specs=[pl.BlockSpec((bn, 1), lambda i: (i, 0)),
                  pl.BlockSpec((1, k), lambda i: (0, 0)),
                  pl.BlockSpec((k, H), lambda i: (0, 0)),
                  pl.BlockSpec((1, H), lambda i: (0, 0))],
        out_specs=pl.BlockSpec((bn, H), lambda i: (i, 0)),
        out_shape=jax.ShapeDtypeStruct((m, H), F32),
    )(angles2d, centers2d, w, b2d)


def _count_body(idxr_ref, idxc_ref, o_ref):
    i = pl.program_id(0)
    q = idxc_ref[...] >> 7
    r = idxr_ref[...] & 127
    ioq = lax.broadcasted_iota(jnp.int32, (NQ, 1), 0)
    ior = lax.broadcasted_iota(jnp.int32, (1, H), 1)
    ohq = (q == ioq).astype(BF16)
    ohr = (r == ior).astype(BF16)
    cblk = jnp.dot(ohq, ohr, preferred_element_type=F32)

    @pl.when(i == 0)
    def _init():
        o_ref[...] = cblk

    @pl.when(i > 0)
    def _accum():
        o_ref[...] += cblk


def _counts(idx):
    n = idx.shape[0]
    bn = 12800
    return pl.pallas_call(
        _count_body,
        grid=(n // bn,),
        in_specs=[pl.BlockSpec((bn, 1), lambda i: (i, 0)),
                  pl.BlockSpec((1, bn), lambda i: (0, i))],
        out_specs=pl.BlockSpec((NQ, H), lambda i: (0, 0)),
        out_shape=jax.ShapeDtypeStruct((NQ, H), F32),
    )(idx.reshape(n, 1), idx.reshape(1, n))


def _sc_body(p_hbm, row_hbm, col_hbm, cen_hbm, be_hbm, bt_hbm,
             oute_hbm, outt_hbm,
             row0_v, row1_v, col0_v, col1_v,
             rp0_v, rp1_v, cp0_v, cp1_v,
             g10_v, g11_v, g20_v, g21_v, b0_v, b1_v,
             acc, sg10, sg11, sg20, sg21, sb0, sb1, sc0, sc1, si0, si1):
    c = lax.axis_index("c")
    s = lax.axis_index("s")
    w = s * NC + c

    row_v = (row0_v, row1_v)
    col_v = (col0_v, col1_v)
    rp_v = (rp0_v, rp1_v)
    cp_v = (cp0_v, cp1_v)
    si = (si0, si1)
    g1_v = (g10_v, g11_v)
    g2_v = (g20_v, g21_v)
    b_v = (b0_v, b1_v)
    sg1 = (sg10, sg11)
    sg2 = (sg20, sg21)
    sb = (sb0, sb1)
    sc = (sc0, sc1)

    n_echunks = row_hbm.shape[0]
    n_tchunks = cen_hbm.shape[0]
    n_e = (n_echunks - 1 - w) // NW + 1
    n_t = (n_tchunks - 1 - w) // NW + 1

    def _zero_g1():
        def _zrow(r, _):
            for blk in range(H // 16):
                g10_v[r, pl.ds(16 * blk, 16)] = jnp.zeros((16,), F32)
            return _
        lax.fori_loop(0, CK, _zrow, None)

    _zero_g1()
    for off, sz in ZSLICES:
        pltpu.sync_copy(g10_v.at[pl.ds(0, sz)],
                        acc.at[pl.ds(s * ROWS_PER_TILE + off, sz)])
    plsc.subcore_barrier()

    def _eissue(k, i):
        ch = w + i * NW
        pltpu.make_async_copy(row_hbm.at[ch], rp_v[k], si[k]).wait()
        pltpu.make_async_copy(col_hbm.at[ch], cp_v[k], si[k]).wait()
        for blk in range(CK // 16):
            sl = pl.ds(16 * blk, 16)
            row_v[k][0, sl] = rp_v[k][0, sl]
            col_v[k][0, sl] = cp_v[k][0, sl]

        @pl.when(i + 2 < n_e)
        def _():
            ch2 = w + (i + 2) * NW
            pltpu.async_copy(row_hbm.at[ch2], rp_v[k], si[k])
            pltpu.async_copy(col_hbm.at[ch2], cp_v[k], si[k])
        pltpu.async_copy(p_hbm.at[row_v[k].at[0]], g1_v[k], sg1[k])
        pltpu.async_copy(p_hbm.at[col_v[k].at[0]], g2_v[k], sg2[k])
        pltpu.async_copy(be_hbm.at[ch], b_v[k], sb[k])

    def _edrain(k):
        pltpu.make_async_copy(p_hbm.at[pl.ds(0, CK)], g1_v[k], sg1[k]).wait()
        pltpu.make_async_copy(p_hbm.at[pl.ds(0, CK)], g2_v[k], sg2[k]).wait()
        pltpu.make_async_copy(p_hbm.at[pl.ds(0, CK)], b_v[k], sb[k]).wait()

    def _ecompute(k):
        def _crow(r4, __):
            for dr in range(4):
                r = r4 * 4 + dr
                for blk in range(H // 16):
                    sl = pl.ds(16 * blk, 16)
                    g1_v[k][r, sl] = _silu(g1_v[k][r, sl] + g2_v[k][r, sl]
                                           + b_v[k][r, sl])
            return __
        lax.fori_loop(0, CK // 4, _crow, None)
        pltpu.async_copy(g1_v[k], acc.at[row_v[k].at[0]], sc[k], add=True)

    def _scwait(k):
        pltpu.make_async_copy(g1_v[k], acc.at[pl.ds(0, CK)], sc[k]).wait()

    pltpu.async_copy(row_hbm.at[w], rp0_v, si0)
    pltpu.async_copy(col_hbm.at[w], cp0_v, si0)

    @pl.when(1 < n_e)
    def _():
        pltpu.async_copy(row_hbm.at[w + NW], rp1_v, si1)
        pltpu.async_copy(col_hbm.at[w + NW], cp1_v, si1)
    _eissue(0, 0)

    def _epair(j, _):
        @pl.when(2 * j + 1 < n_e)
        def _():
            @pl.when(j > 0)
            def _():
                _scwait(1)
            _eissue(1, 2 * j + 1)
        _edrain(0)
        _ecompute(0)

        @pl.when(2 * j + 1 < n_e)
        def _():
            _edrain(1)
            _ecompute(1)
        _scwait(0)

        @pl.when(2 * j + 2 < n_e)
        def _():
            _eissue(0, 2 * j + 2)
        return _
    lax.fori_loop(0, (n_e + 1) // 2, _epair, None)
    _scwait(1)
    plsc.subcore_barrier()

    _zero_g1()
    for off, sz in ZSLICES:
        base = s * ROWS_PER_TILE + off
        pltpu.sync_copy(acc.at[pl.ds(base, sz)],
                        oute_hbm.at[c, pl.ds(base, sz)])
        pltpu.sync_copy(g10_v.at[pl.ds(0, sz)], acc.at[pl.ds(base, sz)])
    plsc.subcore_barrier()

    def _tissue(k, i):
        ch = w + i * NW
        pltpu.make_async_copy(cen_hbm.at[ch], rp_v[k], si[k]).wait()
        for blk in range(CK // 16):
            sl = pl.ds(16 * blk, 16)
            row_v[k][0, sl] = rp_v[k][0, sl]
            col_v[k][0, sl] = rp_v[k][0, sl] + jnp.full((16,), 2 * N_NODES,
                                                        jnp.int32)

        @pl.when(i + 2 < n_t)
        def _():
            pltpu.async_copy(cen_hbm.at[w + (i + 2) * NW], rp_v[k], si[k])
        pltpu.async_copy(p_hbm.at[col_v[k].at[0]], g1_v[k], sg1[k])
        pltpu.async_copy(bt_hbm.at[ch], b_v[k], sb[k])

    def _tdrain(k):
        pltpu.make_async_copy(p_hbm.at[pl.ds(0, CK)], g1_v[k], sg1[k]).wait()
        pltpu.make_async_copy(p_hbm.at[pl.ds(0, CK)], b_v[k], sb[k]).wait()

    def _tcompute(k):
        def _crow(r4, __):
            for dr in range(4):
                r = r4 * 4 + dr
                for blk in range(H // 16):
                    sl = pl.ds(16 * blk, 16)
                    g1_v[k][r, sl] = _silu(g1_v[k][r, sl] + b_v[k][r, sl])
            return __
        lax.fori_loop(0, CK // 4, _crow, None)
        pltpu.async_copy(g1_v[k], acc.at[row_v[k].at[0]], sc[k], add=True)

    pltpu.async_copy(cen_hbm.at[w], rp0_v, si0)

    @pl.when(1 < n_t)
    def _():
        pltpu.async_copy(cen_hbm.at[w + NW], rp1_v, si1)
    _tissue(0, 0)

    def _tpair(j, _):
        @pl.when(2 * j + 1 < n_t)
        def _():
            @pl.when(j > 0)
            def _():
                _scwait(1)
            _tissue(1, 2 * j + 1)
        _tdrain(0)
        _tcompute(0)

        @pl.when(2 * j + 1 < n_t)
        def _():
            _tdrain(1)
            _tcompute(1)
        _scwait(0)

        @pl.when(2 * j + 2 < n_t)
        def _():
            _tissue(0, 2 * j + 2)
        return _
    lax.fori_loop(0, (n_t + 1) // 2, _tpair, None)
    _scwait(1)
    plsc.subcore_barrier()

    for off, sz in ZSLICES:
        base = s * ROWS_PER_TILE + off
        pltpu.sync_copy(acc.at[pl.ds(base, sz)],
                        outt_hbm.at[c, pl.ds(base, sz)])


def _sc_aggregate(p, row3, col3, cen3, be3, bt3):
    mesh = plsc.VectorSubcoreMesh(core_axis_name="c", subcore_axis_name="s")
    outh = jax.ShapeDtypeStruct((NC, N_PAD, H), F32)
    run = pl.kernel(
        _sc_body,
        mesh=mesh,
        out_type=[outh, outh],
        scratch_types=(
            [pltpu.VMEM((1, CK), jnp.int32)] * 8
            + [pltpu.VMEM((CK, H), F32)] * 6
            + [pltpu.VMEM_SHARED((N_PAD, H), F32)]
            + [pltpu.SemaphoreType.DMA] * 10
        ),
    )
    return run(p, row3, col3, cen3, be3, bt3)


def _combine_body(x_ref, se_ref, st_ref, ce_ref, ct_ref,
                  we2_ref, be2_ref, wt2_ref, bt2_ref,
                  wn1a_ref, wn1b_ref, bn1_ref, wn2_ref, bn2_ref, o_ref):
    xb = x_ref[...]
    se = se_ref[0] + se_ref[1]
    st = st_ref[0] + st_ref[1]
    aggr = (jnp.dot(se, we2_ref[...], preferred_element_type=F32, precision=_PREC)
            + ce_ref[...] * be2_ref[...]
            + jnp.dot(st, wt2_ref[...], preferred_element_type=F32, precision=_PREC)
            + ct_ref[...] * bt2_ref[...])
    h1 = (jnp.dot(xb, wn1a_ref[...], preferred_element_type=F32, precision=_PREC)
          + jnp.dot(aggr, wn1b_ref[...], preferred_element_type=F32, precision=_PREC)
          + bn1_ref[...])
    h1 = _silu(h1)
    o_ref[...] = xb + jnp.dot(h1, wn2_ref[...], preferred_element_type=F32,
                              precision=_PREC) + bn2_ref[...]


def _combine(x, se, st, ce, ct, we2, be2, wt2, bt2, wn1a, wn1b, bn1, wn2, bn2):
    n = x.shape[0]
    bn = 2000
    full = lambda r, c: pl.BlockSpec((r, c), lambda i: (0, 0))
    acc_spec = pl.BlockSpec((NC, bn, H), lambda i: (0, i, 0))
    cnt_spec = pl.BlockSpec((bn, 1), lambda i: (i, 0))
    return pl.pallas_call(
        _combine_body,
        grid=(n // bn,),
        in_specs=[pl.BlockSpec((bn, H), lambda i: (i, 0)),
                  acc_spec, acc_spec, cnt_spec, cnt_spec,
                  full(H, H), full(1, H), full(H, H), full(1, H),
                  full(H, H), full(H, H), full(1, H), full(H, H), full(1, H)],
        out_specs=pl.BlockSpec((bn, H), lambda i: (i, 0)),
        out_shape=jax.ShapeDtypeStruct((n, H), F32),
    )(x, se, st, ce, ct, we2, be2, wt2, bt2, wn1a, wn1b, bn1, wn2, bn2)


def kernel(x, edge_index, edge_attr_rbf, triplet_index, angles,
           W_e1, b_e1, W_e2, b_e2,
           W_t1, b_t1, W_t2, b_t2,
           W_n1, b_n1, W_n2, b_n2,
           centers):
    e = edge_index.shape[1]
    m = triplet_index.shape[0]
    k = centers.shape[0]

    w3 = jnp.stack([W_e1[:H], W_e1[H:2 * H], W_t1[:H]])
    p = _node_projections(x, w3)
    be = _edge_rbf_term(edge_attr_rbf, W_e1[2 * H:], b_e1.reshape(1, H))
    bt = _angle_rbf_term(angles.reshape(m, 1), centers.reshape(1, k),
                         W_t1[H:], b_t1.reshape(1, H))

    row = edge_index[0]
    cen = triplet_index[:, 1]
    cnt_e = _counts(row).reshape(NQ * H, 1)
    cnt_t = _counts(cen).reshape(NQ * H, 1)

    n = x.shape[0]
    row3 = row.reshape(e // CK, 1, CK)
    col3 = (edge_index[1] + n).reshape(e // CK, 1, CK)
    cen3 = cen.reshape(m // CK, 1, CK)
    be3 = be.reshape(e // CK, CK, H)
    bt3 = bt.reshape(m // CK, CK, H)
    se, st = _sc_aggregate(p, row3, col3, cen3, be3, bt3)

    return _combine(x, se, st, cnt_e[:N_NODES], cnt_t[:N_NODES],
                    W_e2, b_e2.reshape(1, H), W_t2, b_t2.reshape(1, H),
                    W_n1[:H], W_n1[H:], b_n1.reshape(1, H), W_n2, b_n2.reshape(1, H))
